# Initial kernel scaffold; baseline (speedup 1.0000x reference)
import jax
import jax.numpy as jnp
from jax.experimental import pallas as pl


def kernel(x, w1, b1, w2, b2, fc1_w, fc1_b, fc2_w, fc2_b):
    raise NotImplementedError("write your pallas kernel here")


# speedup vs baseline: 1.3258x; 1.3258x over previous
import functools

import jax
import jax.numpy as jnp
from jax.experimental import pallas as pl
from jax.experimental.pallas import tpu as pltpu

N = 256
KNN = 9


def _topk_idx(dist):
    iota = jax.lax.broadcasted_iota(jnp.int32, (N, N), 1)
    cols = []
    for _ in range(KNN):
        mv = jnp.min(dist, axis=1, keepdims=True)
        mi = jnp.min(jnp.where(dist <= mv, iota, N), axis=1, keepdims=True)
        cols.append(mi)
        dist = jnp.where(iota == mi, jnp.float32(jnp.inf), dist)
    return cols


def _dist(ftb_ref, sq_ref):
    ft = ftb_ref[0]
    sqv = sq_ref[0]
    dot = jax.lax.dot_general(ft, ft, (((1,), (1,)), ((), ())),
                              preferred_element_type=jnp.float32)
    return ft, (jnp.transpose(sqv) + sqv) - 2.0 * dot


def _sel_kernel(ftb_ref, sq_ref, idx_ref):
    _, dist = _dist(ftb_ref, sq_ref)
    idx_ref[0] = jnp.concatenate(_topk_idx(dist), axis=1)


def _conv_kernel(ftb_ref, sq_ref, wg_ref, bias_ref, out_ref, *, o, cu):
    ft, dist = _dist(ftb_ref, sq_ref)
    ftf = ft.astype(jnp.float32)
    iota = jax.lax.broadcasted_iota(jnp.int32, (N, N), 1)
    gs = []
    for mi in _topk_idx(dist):
        onehot = (iota == mi).astype(jnp.float32)
        gs.append(jax.lax.dot_general(
            onehot, ftf, (((1,), (0,)), ((), ())),
            preferred_element_type=jnp.float32)[:, :cu].astype(jnp.bfloat16))
    g = jnp.stack(gs, axis=2).reshape(N, cu * KNN)
    out = jax.lax.dot_general(g, wg_ref[...], (((1,), (0,)), ((), ())),
                              preferred_element_type=jnp.float32)
    out_ref[0] = jnp.maximum(out + bias_ref[...], 0.0)


def _prep(flat, cp):
    c = flat.shape[1]
    ftb = jnp.pad(flat.astype(jnp.bfloat16).transpose(0, 2, 1),
                  ((0, 0), (0, 0), (0, cp - c)))
    sq = jnp.sum(flat * flat, axis=1)[:, None, :]
    return ftb, sq


def _fc_kernel(h_ref, w1_ref, b1_ref, w2_ref, b2_ref, out_ref, acc_ref, *,
               nk):
    kk = pl.program_id(0)

    @pl.when(kk == 0)
    def _init():
        acc_ref[...] = jnp.zeros_like(acc_ref)

    acc_ref[...] += jax.lax.dot_general(
        h_ref[...].astype(jnp.bfloat16), w1_ref[...].astype(jnp.bfloat16),
        (((1,), (1,)), ((), ())), preferred_element_type=jnp.float32)

    @pl.when(kk == nk - 1)
    def _final():
        y = jnp.maximum(acc_ref[...] + b1_ref[...], 0.0)
        out_ref[...] = jax.lax.dot_general(
            y.astype(jnp.bfloat16), w2_ref[...].astype(jnp.bfloat16),
            (((1,), (1,)), ((), ())),
            preferred_element_type=jnp.float32) + b2_ref[...]


def kernel(x, w1, b1, w2, b2, fc1_w, fc1_b, fc2_w, fc2_b):
    B = x.shape[0]
    xu = (x.reshape(B, 3, 16, 2, 16, 2)
           .transpose(0, 1, 3, 5, 2, 4)
           .reshape(B, 12, N))
    g = jnp.linspace(-1.0, 1.0, 16)
    yy, xx = jnp.meshgrid(g, g, indexing='ij')
    ct = jnp.stack([yy, xx]).astype(jnp.float32).reshape(2, N)
    ctb = jnp.broadcast_to(ct[None], (B, 2, N))

    flat1 = jnp.concatenate([xu, ctb], axis=1)
    ftb1, sq1 = _prep(flat1, 16)
    idx1 = pl.pallas_call(
        _sel_kernel,
        grid=(B,),
        in_specs=[
            pl.BlockSpec((1, N, 16), lambda i: (i, 0, 0)),
            pl.BlockSpec((1, 1, N), lambda i: (i, 0, 0)),
        ],
        out_specs=pl.BlockSpec((1, N, KNN), lambda i: (i, 0, 0)),
        out_shape=jax.ShapeDtypeStruct((B, N, KNN), jnp.int32),
    )(ftb1, sq1)
    xt1 = flat1.astype(jnp.bfloat16).transpose(0, 2, 1)
    nb1 = jax.vmap(lambda t, i: t[i])(xt1, idx1)
    h1 = jax.nn.relu(
        jnp.einsum('bnkc,ock->bon', nb1, w1.astype(jnp.bfloat16),
                   preferred_element_type=jnp.float32) + b1[None, :, None])

    flat2 = jnp.concatenate([h1, ctb], axis=1)
    ftb2, sq2 = _prep(flat2, 72)
    wg2 = w2.astype(jnp.bfloat16).reshape(128, 66 * KNN).T
    h2 = pl.pallas_call(
        functools.partial(_conv_kernel, o=128, cu=66),
        grid=(B,),
        in_specs=[
            pl.BlockSpec((1, N, 72), lambda i: (i, 0, 0)),
            pl.BlockSpec((1, 1, N), lambda i: (i, 0, 0)),
            pl.BlockSpec((66 * KNN, 128), lambda i: (0, 0)),
            pl.BlockSpec((1, 128), lambda i: (0, 0)),
        ],
        out_specs=pl.BlockSpec((1, N, 128), lambda i: (i, 0, 0)),
        out_shape=jax.ShapeDtypeStruct((B, N, 128), jnp.float32),
    )(ftb2, sq2, wg2, b2.reshape(1, 128))

    h = (h2.transpose(0, 2, 1)
         .reshape(B, 32, 2, 2, 16, 16)
         .transpose(0, 1, 4, 2, 5, 3)
         .reshape(B, 32768))

    kb = 2048
    nk = 32768 // kb
    return pl.pallas_call(
        functools.partial(_fc_kernel, nk=nk),
        grid=(nk,),
        in_specs=[
            pl.BlockSpec((B, kb), lambda k: (0, k)),
            pl.BlockSpec((1024, kb), lambda k: (0, k)),
            pl.BlockSpec((1, 1024), lambda k: (0, 0)),
            pl.BlockSpec((10, 1024), lambda k: (0, 0)),
            pl.BlockSpec((1, 10), lambda k: (0, 0)),
        ],
        out_specs=pl.BlockSpec((B, 10), lambda k: (0, 0)),
        out_shape=jax.ShapeDtypeStruct((B, 10), jnp.float32),
        scratch_shapes=[pltpu.VMEM((B, 1024), jnp.float32)],
    )(h, fc1_w, fc1_b.reshape(1, 1024), fc2_w, fc2_b.reshape(1, 10))


# speedup vs baseline: 1.8966x; 1.4305x over previous
import functools

import jax
import jax.numpy as jnp
from jax.experimental import pallas as pl
from jax.experimental.pallas import tpu as pltpu

N = 256
KNN = 9


def _topk_idx(dist):
    iota = jax.lax.broadcasted_iota(jnp.int32, (N, N), 1)
    cols = []
    for _ in range(KNN):
        mv = jnp.min(dist, axis=1, keepdims=True)
        mi = jnp.min(jnp.where(dist <= mv, iota, N), axis=1, keepdims=True)
        cols.append(mi)
        dist = jnp.where(iota == mi, jnp.float32(jnp.inf), dist)
    return cols


def _dist(ftb_ref, sq_ref):
    ft = ftb_ref[0]
    sqv = sq_ref[0]
    dot = jax.lax.dot_general(ft, ft, (((1,), (1,)), ((), ())),
                              preferred_element_type=jnp.float32)
    return ft, (jnp.transpose(sqv) + sqv) - 2.0 * dot


def _sel_kernel(ftb_ref, sq_ref, idx_ref):
    _, dist = _dist(ftb_ref, sq_ref)
    idx_ref[0] = jnp.concatenate(_topk_idx(dist), axis=1)


def _conv_kernel(ftb_ref, sq_ref, wg_ref, bias_ref, out_ref, *, o):
    ft, dist = _dist(ftb_ref, sq_ref)
    iota = jax.lax.broadcasted_iota(jnp.int32, (N, N), 1)
    gs = []
    for mi in _topk_idx(dist):
        onehot = (iota == mi).astype(jnp.bfloat16)
        gs.append(jax.lax.dot_general(
            onehot, ft, (((1,), (0,)), ((), ())),
            preferred_element_type=jnp.float32).astype(jnp.bfloat16))
    g = jnp.concatenate(gs, axis=1)
    out = jax.lax.dot_general(g, wg_ref[...], (((1,), (0,)), ((), ())),
                              preferred_element_type=jnp.float32)
    out_ref[0] = jnp.maximum(out + bias_ref[...], 0.0)


def _prep(flat, cp):
    c = flat.shape[1]
    ftb = jnp.pad(flat.astype(jnp.bfloat16).transpose(0, 2, 1),
                  ((0, 0), (0, 0), (0, cp - c)))
    sq = jnp.sum(flat * flat, axis=1)[:, None, :]
    return ftb, sq


def _fc_kernel(h_ref, w1_ref, b1_ref, w2_ref, b2_ref, out_ref, acc_ref, *,
               nk):
    kk = pl.program_id(0)

    @pl.when(kk == 0)
    def _init():
        acc_ref[...] = jnp.zeros_like(acc_ref)

    acc_ref[...] += jax.lax.dot_general(
        h_ref[...].astype(jnp.bfloat16), w1_ref[...].astype(jnp.bfloat16),
        (((1,), (1,)), ((), ())), preferred_element_type=jnp.float32)

    @pl.when(kk == nk - 1)
    def _final():
        y = jnp.maximum(acc_ref[...] + b1_ref[...], 0.0)
        out_ref[...] = jax.lax.dot_general(
            y.astype(jnp.bfloat16), w2_ref[...].astype(jnp.bfloat16),
            (((1,), (1,)), ((), ())),
            preferred_element_type=jnp.float32) + b2_ref[...]


def kernel(x, w1, b1, w2, b2, fc1_w, fc1_b, fc2_w, fc2_b):
    B = x.shape[0]
    xu = (x.reshape(B, 3, 16, 2, 16, 2)
           .transpose(0, 1, 3, 5, 2, 4)
           .reshape(B, 12, N))
    g = jnp.linspace(-1.0, 1.0, 16)
    yy, xx = jnp.meshgrid(g, g, indexing='ij')
    ct = jnp.stack([yy, xx]).astype(jnp.float32).reshape(2, N)
    ctb = jnp.broadcast_to(ct[None], (B, 2, N))

    flat1 = jnp.concatenate([xu, ctb], axis=1)
    ftb1, sq1 = _prep(flat1, 16)
    idx1 = pl.pallas_call(
        _sel_kernel,
        grid=(B,),
        in_specs=[
            pl.BlockSpec((1, N, 16), lambda i: (i, 0, 0)),
            pl.BlockSpec((1, 1, N), lambda i: (i, 0, 0)),
        ],
        out_specs=pl.BlockSpec((1, N, KNN), lambda i: (i, 0, 0)),
        out_shape=jax.ShapeDtypeStruct((B, N, KNN), jnp.int32),
    )(ftb1, sq1)
    xt1 = flat1.astype(jnp.bfloat16).transpose(0, 2, 1)
    nb1 = jax.vmap(lambda t, i: t[i])(xt1, idx1)
    h1 = jax.nn.relu(
        jnp.einsum('bnkc,ock->bon', nb1, w1.astype(jnp.bfloat16),
                   preferred_element_type=jnp.float32) + b1[None, :, None])

    flat2 = jnp.concatenate([h1, ctb], axis=1)
    ftb2, sq2 = _prep(flat2, 72)
    wg2 = (jnp.pad(w2.astype(jnp.bfloat16), ((0, 0), (0, 6), (0, 0)))
           .transpose(2, 1, 0).reshape(KNN * 72, 128))
    h2 = pl.pallas_call(
        functools.partial(_conv_kernel, o=128),
        grid=(B,),
        in_specs=[
            pl.BlockSpec((1, N, 72), lambda i: (i, 0, 0)),
            pl.BlockSpec((1, 1, N), lambda i: (i, 0, 0)),
            pl.BlockSpec((KNN * 72, 128), lambda i: (0, 0)),
            pl.BlockSpec((1, 128), lambda i: (0, 0)),
        ],
        out_specs=pl.BlockSpec((1, N, 128), lambda i: (i, 0, 0)),
        out_shape=jax.ShapeDtypeStruct((B, N, 128), jnp.float32),
    )(ftb2, sq2, wg2, b2.reshape(1, 128))

    h = (h2.transpose(0, 2, 1)
         .reshape(B, 32, 2, 2, 16, 16)
         .transpose(0, 1, 4, 2, 5, 3)
         .reshape(B, 32768))

    kb = 2048
    nk = 32768 // kb
    return pl.pallas_call(
        functools.partial(_fc_kernel, nk=nk),
        grid=(nk,),
        in_specs=[
            pl.BlockSpec((B, kb), lambda k: (0, k)),
            pl.BlockSpec((1024, kb), lambda k: (0, k)),
            pl.BlockSpec((1, 1024), lambda k: (0, 0)),
            pl.BlockSpec((10, 1024), lambda k: (0, 0)),
            pl.BlockSpec((1, 10), lambda k: (0, 0)),
        ],
        out_specs=pl.BlockSpec((B, 10), lambda k: (0, 0)),
        out_shape=jax.ShapeDtypeStruct((B, 10), jnp.float32),
        scratch_shapes=[pltpu.VMEM((B, 1024), jnp.float32)],
    )(h, fc1_w, fc1_b.reshape(1, 1024), fc2_w, fc2_b.reshape(1, 10))
